# baseline (speedup 1.0000x reference)
import jax
import jax.numpy as jnp
from jax import lax
from jax.experimental import pallas as pl
from jax.experimental.pallas import tpu as pltpu
from jax.experimental.pallas import tpu_sc as plsc

D = 256
E = 16384
G = 101
TAB = G * G
TABP = TAB + 7
NC = 2
NS = 16
NW = NC * NS
DPW = D // NW
L = 16
CHUNKS = E // L
JV = D // L
RADIUS = 50


def _sc_body(ev_hbm, t0_hbm, t1_hbm, ct_hbm, calib_hbm,
             c0_hbm, c1_hbm, m_hbm, pd_hbm, corr_hbm,
             new0_hbm, new1_hbm, ne_hbm,
             fex_v, fey_v, t0_v, t1_v, ct_v, calib_v,
             c0_v, c1_v, m_v, pd_v, corr_v,
             n0_v, n1_v, ne_v, sem_big, sem_small):
    c = lax.axis_index("c")
    s = lax.axis_index("s")
    wid = s * NC + c

    big = [
        pltpu.async_copy(ev_hbm.at[0], fex_v, sem_big),
        pltpu.async_copy(ev_hbm.at[1], fey_v, sem_big),
        pltpu.async_copy(t0_hbm, t0_v, sem_big),
        pltpu.async_copy(t1_hbm, t1_v, sem_big),
        pltpu.async_copy(ct_hbm, ct_v, sem_big),
    ]
    small = [
        pltpu.async_copy(
            calib_hbm.at[pl.ds(wid * (2 * DPW), 2 * DPW)], calib_v,
            sem_small),
        pltpu.async_copy(c0_hbm, c0_v, sem_small),
        pltpu.async_copy(c1_hbm, c1_v, sem_small),
        pltpu.async_copy(m_hbm.at[pl.ds(wid * DPW, DPW), :], m_v, sem_small),
        pltpu.async_copy(pd_hbm.at[pl.ds(wid * DPW, DPW), :], pd_v,
                         sem_small),
        pltpu.async_copy(corr_hbm, corr_v, sem_small),
    ]
    for h in small:
        h.wait()

    cvec = calib_v[...]
    cxs = [cvec[2 * d + 1] for d in range(DPW)]
    cys = [cvec[2 * d] for d in range(DPW)]

    zf = jnp.zeros((L,), jnp.float32)
    zi = jnp.zeros((L,), jnp.int32)

    def reg_body(jv, carry):
        rac = list(carry)
        c1j = c1_v[pl.ds(jv * L, L)]
        c0j = c0_v[pl.ds(jv * L, L)]
        for d in range(DPW):
            mj = m_v[d, pl.ds(jv * L, L)]
            pj = pd_v[d, pl.ds(jv * L, L)]
            dxc = c1j - cxs[d]
            dyc = c0j - cys[d]
            sdx = dxc * mj
            sdy = dyc * mj
            radi = sdx * sdx + sdy * sdy - pj * pj
            rac[d] = rac[d] + dxc * radi
            rac[DPW + d] = rac[DPW + d] + dyc * radi
        return tuple(rac)

    rac = lax.fori_loop(0, JV, reg_body, tuple([zf] * (2 * DPW)))

    lanes = lax.iota(jnp.int32, L)
    cdxv = zf
    cdyv = zf
    c0d = zf
    c1d = zf
    for d in range(DPW):
        cdxv = jnp.where(lanes == d, jnp.sum(rac[d]), cdxv)
        cdyv = jnp.where(lanes == d, jnp.sum(rac[DPW + d]), cdyv)
        c0d = jnp.where(lanes == d, cvec[2 * d], c0d)
        c1d = jnp.where(lanes == d, cvec[2 * d + 1], c1d)

    for h in big:
        h.wait()

    init = tuple([zf] * DPW + [zf] * DPW + [zi] * DPW)

    @plsc.parallel_loop(0, CHUNKS, step=1, unroll=2, carry=init)
    def accs(i, carry):
        accs = list(carry)
        ex = fex_v[pl.ds(i * L, L)]
        ey = fey_v[pl.ds(i * L, L)]
        fr = jnp.float32(RADIUS)
        for d in range(DPW):
            ix = jnp.clip(ex - cxs[d], -fr, fr).astype(jnp.int32)
            iy = jnp.clip(ey - cys[d], -fr, fr).astype(jnp.int32)
            flat = ix * G + (iy + (RADIUS * G + RADIUS))
            g0 = plsc.load_gather(t0_v, [flat])
            g1 = plsc.load_gather(t1_v, [flat])
            gc = plsc.load_gather(ct_v, [flat])
            accs[d] = accs[d] + g0
            accs[DPW + d] = accs[DPW + d] + g1
            accs[2 * DPW + d] = accs[2 * DPW + d] + gc
        return tuple(accs)

    o0 = zf
    o1 = zf
    oc = zi
    for d in range(DPW):
        o0 = jnp.where(lanes == d, jnp.sum(accs[d]), o0)
        o1 = jnp.where(lanes == d, jnp.sum(accs[DPW + d]), o1)
        oc = jnp.where(lanes == d, jnp.sum(accs[2 * DPW + d]), oc)

    dec = ((oc & 0xFFFF) >= 10).astype(jnp.float32)
    nev = lax.shift_right_arithmetic(oc, 16)
    lr = jnp.float32(200 * 1.5e-05)
    rfc = corr_v[...] * jnp.float32(4.0 * 1.0 * 2.5e-07)
    new1 = c1d - lr * dec * (jnp.clip(o0, -400.0, 400.0) - rfc * cdxv)
    new0 = c0d - lr * dec * (jnp.clip(o1, -400.0, 400.0) - rfc * cdyv)
    n0_v[...] = new0
    n1_v[...] = new1
    ne_v[...] = nev
    pltpu.sync_copy(n0_v, new0_hbm.at[pl.ds(wid * L, L)])
    pltpu.sync_copy(n1_v, new1_hbm.at[pl.ds(wid * L, L)])
    pltpu.sync_copy(ne_v, ne_hbm.at[pl.ds(wid * L, L)])


_sc_all = pl.kernel(
    _sc_body,
    out_type=[
        jax.ShapeDtypeStruct((NW * L,), jnp.float32),
        jax.ShapeDtypeStruct((NW * L,), jnp.float32),
        jax.ShapeDtypeStruct((NW * L,), jnp.int32),
    ],
    mesh=plsc.VectorSubcoreMesh(core_axis_name="c", subcore_axis_name="s"),
    compiler_params=pltpu.CompilerParams(needs_layout_passes=False),
    scratch_types=[
        pltpu.VMEM((E,), jnp.float32),
        pltpu.VMEM((E,), jnp.float32),
        pltpu.VMEM((TABP,), jnp.float32),
        pltpu.VMEM((TABP,), jnp.float32),
        pltpu.VMEM((TABP,), jnp.int32),
        pltpu.VMEM((2 * DPW,), jnp.float32),
        pltpu.VMEM((D,), jnp.float32),
        pltpu.VMEM((D,), jnp.float32),
        pltpu.VMEM((DPW, D), jnp.float32),
        pltpu.VMEM((DPW, D), jnp.float32),
        pltpu.VMEM((L,), jnp.float32),
        pltpu.VMEM((L,), jnp.float32),
        pltpu.VMEM((L,), jnp.float32),
        pltpu.VMEM((L,), jnp.int32),
        pltpu.SemaphoreType.DMA,
        pltpu.SemaphoreType.DMA,
    ],
)


def kernel(events_x, events_y, calib_center, precompute_grid,
           pairwise_dists_mask, pairwise_dists, correction):
    fex = events_x.astype(jnp.float32)
    fey = events_y.astype(jnp.float32)
    t0 = precompute_grid[:, :, 0].reshape(-1)
    t1 = precompute_grid[:, :, 1].reshape(-1)
    nz = (t0 != 0).astype(jnp.int32) + (t1 != 0).astype(jnp.int32)
    ii = jnp.arange(G, dtype=jnp.int32)
    interior = jnp.logical_and(ii >= 1, ii <= G - 2)
    vic = jnp.logical_and(interior[:, None], interior[None, :])
    ct = nz + (vic.reshape(-1).astype(jnp.int32) << 16)
    t0p = jnp.pad(t0, (0, TABP - TAB))
    t1p = jnp.pad(t1, (0, TABP - TAB))
    ctp = jnp.pad(ct, (0, TABP - TAB))
    corrv = jnp.full((L,), correction, jnp.float32)
    ev = jnp.stack([fex, fey])

    new0, new1, ne = _sc_all(
        ev, t0p, t1p, ctp, calib_center.reshape(2 * D),
        calib_center[:, 0], calib_center[:, 1],
        pairwise_dists_mask, pairwise_dists, corrv)
    new0 = new0.reshape(NW, L)[:, :DPW].reshape(D)
    new1 = new1.reshape(NW, L)[:, :DPW].reshape(D)
    ne = ne.reshape(NW, L)[:, :DPW].reshape(D)
    calib_out = jnp.stack([new0, new1], axis=1)
    return (calib_out, ne)


# speedup vs baseline: 1.0222x; 1.0222x over previous
import jax
import jax.numpy as jnp
from jax import lax
from jax.experimental import pallas as pl
from jax.experimental.pallas import tpu as pltpu
from jax.experimental.pallas import tpu_sc as plsc

D = 256
E = 16384
G = 101
TAB = G * G
TABP = TAB + 7
NC = 2
NS = 16
NW = NC * NS
DPW = D // NW
L = 16
CHUNKS = E // L
JV = D // L
RADIUS = 50


def _sc_body(ev_hbm, t0_hbm, t1_hbm, ct_hbm, calib_hbm,
             c0_hbm, c1_hbm, m_hbm, pd_hbm, corr_hbm,
             new0_hbm, new1_hbm, ne_hbm,
             fex_v, fey_v, t0_v, t1_v, ct_v, calib_v,
             c0_v, c1_v, m_v, pd_v, corr_v,
             n0_v, n1_v, ne_v, sem_big, sem_small):
    c = lax.axis_index("c")
    s = lax.axis_index("s")
    wid = s * NC + c

    big = [
        pltpu.async_copy(ev_hbm.at[0], fex_v, sem_big),
        pltpu.async_copy(ev_hbm.at[1], fey_v, sem_big),
        pltpu.async_copy(t0_hbm, t0_v, sem_big),
        pltpu.async_copy(t1_hbm, t1_v, sem_big),
        pltpu.async_copy(ct_hbm, ct_v, sem_big),
    ]
    small = [
        pltpu.async_copy(
            calib_hbm.at[pl.ds(wid * (2 * DPW), 2 * DPW)], calib_v,
            sem_small),
        pltpu.async_copy(c0_hbm, c0_v, sem_small),
        pltpu.async_copy(c1_hbm, c1_v, sem_small),
        pltpu.async_copy(m_hbm.at[pl.ds(wid * DPW, DPW), :], m_v, sem_small),
        pltpu.async_copy(pd_hbm.at[pl.ds(wid * DPW, DPW), :], pd_v,
                         sem_small),
        pltpu.async_copy(corr_hbm, corr_v, sem_small),
    ]
    for h in small:
        h.wait()

    cvec = calib_v[...]
    cxs = [cvec[2 * d + 1] for d in range(DPW)]
    cys = [cvec[2 * d] for d in range(DPW)]

    zf = jnp.zeros((L,), jnp.float32)
    zi = jnp.zeros((L,), jnp.int32)

    def reg_body(jv, carry):
        rac = list(carry)
        c1j = c1_v[pl.ds(jv * L, L)]
        c0j = c0_v[pl.ds(jv * L, L)]
        for d in range(DPW):
            mj = m_v[d, pl.ds(jv * L, L)]
            pj = pd_v[d, pl.ds(jv * L, L)]
            dxc = c1j - cxs[d]
            dyc = c0j - cys[d]
            sdx = dxc * mj
            sdy = dyc * mj
            radi = sdx * sdx + sdy * sdy - pj * pj
            rac[d] = rac[d] + dxc * radi
            rac[DPW + d] = rac[DPW + d] + dyc * radi
        return tuple(rac)

    rac = lax.fori_loop(0, JV, reg_body, tuple([zf] * (2 * DPW)))

    lanes = lax.iota(jnp.int32, L)
    cdxv = zf
    cdyv = zf
    c0d = zf
    c1d = zf
    for d in range(DPW):
        cdxv = jnp.where(lanes == d, jnp.sum(rac[d]), cdxv)
        cdyv = jnp.where(lanes == d, jnp.sum(rac[DPW + d]), cdyv)
        c0d = jnp.where(lanes == d, cvec[2 * d], c0d)
        c1d = jnp.where(lanes == d, cvec[2 * d + 1], c1d)

    for h in big:
        h.wait()

    init = tuple([zf] * DPW + [zf] * DPW + [zi] * DPW)

    @plsc.parallel_loop(0, CHUNKS, step=1, unroll=4, carry=init)
    def accs(i, carry):
        accs = list(carry)
        ex = fex_v[pl.ds(i * L, L)]
        ey = fey_v[pl.ds(i * L, L)]
        fr = jnp.float32(RADIUS)
        for d in range(DPW):
            ix = jnp.clip(ex - cxs[d], -fr, fr).astype(jnp.int32)
            iy = jnp.clip(ey - cys[d], -fr, fr).astype(jnp.int32)
            flat = ix * G + (iy + (RADIUS * G + RADIUS))
            g0 = plsc.load_gather(t0_v, [flat])
            g1 = plsc.load_gather(t1_v, [flat])
            gc = plsc.load_gather(ct_v, [flat])
            accs[d] = accs[d] + g0
            accs[DPW + d] = accs[DPW + d] + g1
            accs[2 * DPW + d] = accs[2 * DPW + d] + gc
        return tuple(accs)

    o0 = zf
    o1 = zf
    oc = zi
    for d in range(DPW):
        o0 = jnp.where(lanes == d, jnp.sum(accs[d]), o0)
        o1 = jnp.where(lanes == d, jnp.sum(accs[DPW + d]), o1)
        oc = jnp.where(lanes == d, jnp.sum(accs[2 * DPW + d]), oc)

    dec = ((oc & 0xFFFF) >= 10).astype(jnp.float32)
    nev = lax.shift_right_arithmetic(oc, 16)
    lr = jnp.float32(200 * 1.5e-05)
    rfc = corr_v[...] * jnp.float32(4.0 * 1.0 * 2.5e-07)
    new1 = c1d - lr * dec * (jnp.clip(o0, -400.0, 400.0) - rfc * cdxv)
    new0 = c0d - lr * dec * (jnp.clip(o1, -400.0, 400.0) - rfc * cdyv)
    n0_v[...] = new0
    n1_v[...] = new1
    ne_v[...] = nev
    pltpu.sync_copy(n0_v, new0_hbm.at[pl.ds(wid * L, L)])
    pltpu.sync_copy(n1_v, new1_hbm.at[pl.ds(wid * L, L)])
    pltpu.sync_copy(ne_v, ne_hbm.at[pl.ds(wid * L, L)])


_sc_all = pl.kernel(
    _sc_body,
    out_type=[
        jax.ShapeDtypeStruct((NW * L,), jnp.float32),
        jax.ShapeDtypeStruct((NW * L,), jnp.float32),
        jax.ShapeDtypeStruct((NW * L,), jnp.int32),
    ],
    mesh=plsc.VectorSubcoreMesh(core_axis_name="c", subcore_axis_name="s"),
    compiler_params=pltpu.CompilerParams(needs_layout_passes=False),
    scratch_types=[
        pltpu.VMEM((E,), jnp.float32),
        pltpu.VMEM((E,), jnp.float32),
        pltpu.VMEM((TABP,), jnp.float32),
        pltpu.VMEM((TABP,), jnp.float32),
        pltpu.VMEM((TABP,), jnp.int32),
        pltpu.VMEM((2 * DPW,), jnp.float32),
        pltpu.VMEM((D,), jnp.float32),
        pltpu.VMEM((D,), jnp.float32),
        pltpu.VMEM((DPW, D), jnp.float32),
        pltpu.VMEM((DPW, D), jnp.float32),
        pltpu.VMEM((L,), jnp.float32),
        pltpu.VMEM((L,), jnp.float32),
        pltpu.VMEM((L,), jnp.float32),
        pltpu.VMEM((L,), jnp.int32),
        pltpu.SemaphoreType.DMA,
        pltpu.SemaphoreType.DMA,
    ],
)


def kernel(events_x, events_y, calib_center, precompute_grid,
           pairwise_dists_mask, pairwise_dists, correction):
    fex = events_x.astype(jnp.float32)
    fey = events_y.astype(jnp.float32)
    t0 = precompute_grid[:, :, 0].reshape(-1)
    t1 = precompute_grid[:, :, 1].reshape(-1)
    nz = (t0 != 0).astype(jnp.int32) + (t1 != 0).astype(jnp.int32)
    ii = jnp.arange(G, dtype=jnp.int32)
    interior = jnp.logical_and(ii >= 1, ii <= G - 2)
    vic = jnp.logical_and(interior[:, None], interior[None, :])
    ct = nz + (vic.reshape(-1).astype(jnp.int32) << 16)
    t0p = jnp.pad(t0, (0, TABP - TAB))
    t1p = jnp.pad(t1, (0, TABP - TAB))
    ctp = jnp.pad(ct, (0, TABP - TAB))
    corrv = jnp.full((L,), correction, jnp.float32)
    ev = jnp.stack([fex, fey])

    new0, new1, ne = _sc_all(
        ev, t0p, t1p, ctp, calib_center.reshape(2 * D),
        calib_center[:, 0], calib_center[:, 1],
        pairwise_dists_mask, pairwise_dists, corrv)
    new0 = new0.reshape(NW, L)[:, :DPW].reshape(D)
    new1 = new1.reshape(NW, L)[:, :DPW].reshape(D)
    ne = ne.reshape(NW, L)[:, :DPW].reshape(D)
    calib_out = jnp.stack([new0, new1], axis=1)
    return (calib_out, ne)


# speedup vs baseline: 1.0231x; 1.0009x over previous
import jax
import jax.numpy as jnp
from jax import lax
from jax.experimental import pallas as pl
from jax.experimental.pallas import tpu as pltpu
from jax.experimental.pallas import tpu_sc as plsc

D = 256
E = 16384
G = 101
TAB = G * G
TABP = TAB + 7
NC = 2
NS = 16
NW = NC * NS
DPW = D // NW
L = 16
CHUNKS = E // L
JV = D // L
RADIUS = 50


def _sc_body(ev_hbm, t0_hbm, t1_hbm, ct_hbm, calib_hbm,
             c0_hbm, c1_hbm, m_hbm, pd_hbm, corr_hbm,
             new0_hbm, new1_hbm, ne_hbm,
             fex_v, fey_v, t0_v, t1_v, ct_v, calib_v,
             c0_v, c1_v, m_v, pd_v, corr_v,
             n0_v, n1_v, ne_v, sem_big, sem_small):
    c = lax.axis_index("c")
    s = lax.axis_index("s")
    wid = s * NC + c

    big = [
        pltpu.async_copy(ev_hbm.at[0], fex_v, sem_big),
        pltpu.async_copy(ev_hbm.at[1], fey_v, sem_big),
        pltpu.async_copy(t0_hbm, t0_v, sem_big),
        pltpu.async_copy(t1_hbm, t1_v, sem_big),
        pltpu.async_copy(ct_hbm, ct_v, sem_big),
    ]
    small = [
        pltpu.async_copy(
            calib_hbm.at[pl.ds(wid * (2 * DPW), 2 * DPW)], calib_v,
            sem_small),
        pltpu.async_copy(c0_hbm, c0_v, sem_small),
        pltpu.async_copy(c1_hbm, c1_v, sem_small),
        pltpu.async_copy(m_hbm.at[pl.ds(wid * DPW, DPW), :], m_v, sem_small),
        pltpu.async_copy(pd_hbm.at[pl.ds(wid * DPW, DPW), :], pd_v,
                         sem_small),
        pltpu.async_copy(corr_hbm, corr_v, sem_small),
    ]
    for h in small:
        h.wait()

    cvec = calib_v[...]
    cxs = [cvec[2 * d + 1] for d in range(DPW)]
    cys = [cvec[2 * d] for d in range(DPW)]

    zf = jnp.zeros((L,), jnp.float32)
    zi = jnp.zeros((L,), jnp.int32)

    def reg_body(jv, carry):
        rac = list(carry)
        c1j = c1_v[pl.ds(jv * L, L)]
        c0j = c0_v[pl.ds(jv * L, L)]
        for d in range(DPW):
            mj = m_v[d, pl.ds(jv * L, L)]
            pj = pd_v[d, pl.ds(jv * L, L)]
            dxc = c1j - cxs[d]
            dyc = c0j - cys[d]
            sdx = dxc * mj
            sdy = dyc * mj
            radi = sdx * sdx + sdy * sdy - pj * pj
            rac[d] = rac[d] + dxc * radi
            rac[DPW + d] = rac[DPW + d] + dyc * radi
        return tuple(rac)

    rac = lax.fori_loop(0, JV, reg_body, tuple([zf] * (2 * DPW)))

    lanes = lax.iota(jnp.int32, L)
    cdxv = zf
    cdyv = zf
    c0d = zf
    c1d = zf
    for d in range(DPW):
        cdxv = jnp.where(lanes == d, jnp.sum(rac[d]), cdxv)
        cdyv = jnp.where(lanes == d, jnp.sum(rac[DPW + d]), cdyv)
        c0d = jnp.where(lanes == d, cvec[2 * d], c0d)
        c1d = jnp.where(lanes == d, cvec[2 * d + 1], c1d)

    for h in big:
        h.wait()

    init = tuple([zf] * DPW + [zf] * DPW + [zi] * DPW)

    @plsc.parallel_loop(0, CHUNKS, step=1, unroll=8, carry=init)
    def accs(i, carry):
        accs = list(carry)
        ex = fex_v[pl.ds(i * L, L)]
        ey = fey_v[pl.ds(i * L, L)]
        fr = jnp.float32(RADIUS)
        for d in range(DPW):
            ix = jnp.clip(ex - cxs[d], -fr, fr).astype(jnp.int32)
            iy = jnp.clip(ey - cys[d], -fr, fr).astype(jnp.int32)
            flat = ix * G + (iy + (RADIUS * G + RADIUS))
            g0 = plsc.load_gather(t0_v, [flat])
            g1 = plsc.load_gather(t1_v, [flat])
            gc = plsc.load_gather(ct_v, [flat])
            accs[d] = accs[d] + g0
            accs[DPW + d] = accs[DPW + d] + g1
            accs[2 * DPW + d] = accs[2 * DPW + d] + gc
        return tuple(accs)

    o0 = zf
    o1 = zf
    oc = zi
    for d in range(DPW):
        o0 = jnp.where(lanes == d, jnp.sum(accs[d]), o0)
        o1 = jnp.where(lanes == d, jnp.sum(accs[DPW + d]), o1)
        oc = jnp.where(lanes == d, jnp.sum(accs[2 * DPW + d]), oc)

    dec = ((oc & 0xFFFF) >= 10).astype(jnp.float32)
    nev = lax.shift_right_arithmetic(oc, 16)
    lr = jnp.float32(200 * 1.5e-05)
    rfc = corr_v[...] * jnp.float32(4.0 * 1.0 * 2.5e-07)
    new1 = c1d - lr * dec * (jnp.clip(o0, -400.0, 400.0) - rfc * cdxv)
    new0 = c0d - lr * dec * (jnp.clip(o1, -400.0, 400.0) - rfc * cdyv)
    n0_v[...] = new0
    n1_v[...] = new1
    ne_v[...] = nev
    pltpu.sync_copy(n0_v, new0_hbm.at[pl.ds(wid * L, L)])
    pltpu.sync_copy(n1_v, new1_hbm.at[pl.ds(wid * L, L)])
    pltpu.sync_copy(ne_v, ne_hbm.at[pl.ds(wid * L, L)])


_sc_all = pl.kernel(
    _sc_body,
    out_type=[
        jax.ShapeDtypeStruct((NW * L,), jnp.float32),
        jax.ShapeDtypeStruct((NW * L,), jnp.float32),
        jax.ShapeDtypeStruct((NW * L,), jnp.int32),
    ],
    mesh=plsc.VectorSubcoreMesh(core_axis_name="c", subcore_axis_name="s"),
    compiler_params=pltpu.CompilerParams(needs_layout_passes=False),
    scratch_types=[
        pltpu.VMEM((E,), jnp.float32),
        pltpu.VMEM((E,), jnp.float32),
        pltpu.VMEM((TABP,), jnp.float32),
        pltpu.VMEM((TABP,), jnp.float32),
        pltpu.VMEM((TABP,), jnp.int32),
        pltpu.VMEM((2 * DPW,), jnp.float32),
        pltpu.VMEM((D,), jnp.float32),
        pltpu.VMEM((D,), jnp.float32),
        pltpu.VMEM((DPW, D), jnp.float32),
        pltpu.VMEM((DPW, D), jnp.float32),
        pltpu.VMEM((L,), jnp.float32),
        pltpu.VMEM((L,), jnp.float32),
        pltpu.VMEM((L,), jnp.float32),
        pltpu.VMEM((L,), jnp.int32),
        pltpu.SemaphoreType.DMA,
        pltpu.SemaphoreType.DMA,
    ],
)


def kernel(events_x, events_y, calib_center, precompute_grid,
           pairwise_dists_mask, pairwise_dists, correction):
    fex = events_x.astype(jnp.float32)
    fey = events_y.astype(jnp.float32)
    t0 = precompute_grid[:, :, 0].reshape(-1)
    t1 = precompute_grid[:, :, 1].reshape(-1)
    nz = (t0 != 0).astype(jnp.int32) + (t1 != 0).astype(jnp.int32)
    ii = jnp.arange(G, dtype=jnp.int32)
    interior = jnp.logical_and(ii >= 1, ii <= G - 2)
    vic = jnp.logical_and(interior[:, None], interior[None, :])
    ct = nz + (vic.reshape(-1).astype(jnp.int32) << 16)
    t0p = jnp.pad(t0, (0, TABP - TAB))
    t1p = jnp.pad(t1, (0, TABP - TAB))
    ctp = jnp.pad(ct, (0, TABP - TAB))
    corrv = jnp.full((L,), correction, jnp.float32)
    ev = jnp.stack([fex, fey])

    new0, new1, ne = _sc_all(
        ev, t0p, t1p, ctp, calib_center.reshape(2 * D),
        calib_center[:, 0], calib_center[:, 1],
        pairwise_dists_mask, pairwise_dists, corrv)
    new0 = new0.reshape(NW, L)[:, :DPW].reshape(D)
    new1 = new1.reshape(NW, L)[:, :DPW].reshape(D)
    ne = ne.reshape(NW, L)[:, :DPW].reshape(D)
    calib_out = jnp.stack([new0, new1], axis=1)
    return (calib_out, ne)


# speedup vs baseline: 1.0247x; 1.0016x over previous
import jax
import jax.numpy as jnp
from jax import lax
from jax.experimental import pallas as pl
from jax.experimental.pallas import tpu as pltpu
from jax.experimental.pallas import tpu_sc as plsc

D = 256
E = 16384
G = 101
TAB = G * G
TABP = TAB + 7
NC = 2
NS = 16
NW = NC * NS
DPW = D // NW
L = 16
CHUNKS = E // L
JV = D // L
RADIUS = 50


def _sc_body(ev_hbm, t0_hbm, t1_hbm, ct_hbm, calib_hbm,
             c0_hbm, c1_hbm, m_hbm, pd_hbm, corr_hbm,
             new0_hbm, new1_hbm, ne_hbm,
             fex_v, fey_v, t0_v, t1_v, ct_v, calib_v,
             c0_v, c1_v, m_v, pd_v, corr_v,
             n0_v, n1_v, ne_v, sem_big, sem_small):
    c = lax.axis_index("c")
    s = lax.axis_index("s")
    wid = s * NC + c

    big = [
        pltpu.async_copy(ev_hbm.at[0], fex_v, sem_big),
        pltpu.async_copy(ev_hbm.at[1], fey_v, sem_big),
        pltpu.async_copy(t0_hbm, t0_v, sem_big),
        pltpu.async_copy(t1_hbm, t1_v, sem_big),
        pltpu.async_copy(ct_hbm, ct_v, sem_big),
    ]
    small = [
        pltpu.async_copy(
            calib_hbm.at[pl.ds(wid * (2 * DPW), 2 * DPW)], calib_v,
            sem_small),
        pltpu.async_copy(c0_hbm, c0_v, sem_small),
        pltpu.async_copy(c1_hbm, c1_v, sem_small),
        pltpu.async_copy(m_hbm.at[pl.ds(wid * DPW, DPW), :], m_v, sem_small),
        pltpu.async_copy(pd_hbm.at[pl.ds(wid * DPW, DPW), :], pd_v,
                         sem_small),
        pltpu.async_copy(corr_hbm, corr_v, sem_small),
    ]
    for h in small:
        h.wait()

    cvec = calib_v[...]
    cxs = [cvec[2 * d + 1] for d in range(DPW)]
    cys = [cvec[2 * d] for d in range(DPW)]

    zf = jnp.zeros((L,), jnp.float32)
    zi = jnp.zeros((L,), jnp.int32)

    def reg_body(jv, carry):
        rac = list(carry)
        c1j = c1_v[pl.ds(jv * L, L)]
        c0j = c0_v[pl.ds(jv * L, L)]
        for d in range(DPW):
            mj = m_v[d, pl.ds(jv * L, L)]
            pj = pd_v[d, pl.ds(jv * L, L)]
            dxc = c1j - cxs[d]
            dyc = c0j - cys[d]
            sdx = dxc * mj
            sdy = dyc * mj
            radi = sdx * sdx + sdy * sdy - pj * pj
            rac[d] = rac[d] + dxc * radi
            rac[DPW + d] = rac[DPW + d] + dyc * radi
        return tuple(rac)

    rac = lax.fori_loop(0, JV, reg_body, tuple([zf] * (2 * DPW)))

    lanes = lax.iota(jnp.int32, L)
    cdxv = zf
    cdyv = zf
    c0d = zf
    c1d = zf
    for d in range(DPW):
        cdxv = jnp.where(lanes == d, jnp.sum(rac[d]), cdxv)
        cdyv = jnp.where(lanes == d, jnp.sum(rac[DPW + d]), cdyv)
        c0d = jnp.where(lanes == d, cvec[2 * d], c0d)
        c1d = jnp.where(lanes == d, cvec[2 * d + 1], c1d)

    for h in big:
        h.wait()

    init = tuple([zf] * DPW + [zf] * DPW + [zi] * DPW)

    @plsc.parallel_loop(0, CHUNKS, step=1, unroll=16, carry=init)
    def accs(i, carry):
        accs = list(carry)
        ex = fex_v[pl.ds(i * L, L)]
        ey = fey_v[pl.ds(i * L, L)]
        fr = jnp.float32(RADIUS)
        for d in range(DPW):
            ix = jnp.clip(ex - cxs[d], -fr, fr).astype(jnp.int32)
            iy = jnp.clip(ey - cys[d], -fr, fr).astype(jnp.int32)
            flat = ix * G + (iy + (RADIUS * G + RADIUS))
            g0 = plsc.load_gather(t0_v, [flat])
            g1 = plsc.load_gather(t1_v, [flat])
            gc = plsc.load_gather(ct_v, [flat])
            accs[d] = accs[d] + g0
            accs[DPW + d] = accs[DPW + d] + g1
            accs[2 * DPW + d] = accs[2 * DPW + d] + gc
        return tuple(accs)

    o0 = zf
    o1 = zf
    oc = zi
    for d in range(DPW):
        o0 = jnp.where(lanes == d, jnp.sum(accs[d]), o0)
        o1 = jnp.where(lanes == d, jnp.sum(accs[DPW + d]), o1)
        oc = jnp.where(lanes == d, jnp.sum(accs[2 * DPW + d]), oc)

    dec = ((oc & 0xFFFF) >= 10).astype(jnp.float32)
    nev = lax.shift_right_arithmetic(oc, 16)
    lr = jnp.float32(200 * 1.5e-05)
    rfc = corr_v[...] * jnp.float32(4.0 * 1.0 * 2.5e-07)
    new1 = c1d - lr * dec * (jnp.clip(o0, -400.0, 400.0) - rfc * cdxv)
    new0 = c0d - lr * dec * (jnp.clip(o1, -400.0, 400.0) - rfc * cdyv)
    n0_v[...] = new0
    n1_v[...] = new1
    ne_v[...] = nev
    pltpu.sync_copy(n0_v, new0_hbm.at[pl.ds(wid * L, L)])
    pltpu.sync_copy(n1_v, new1_hbm.at[pl.ds(wid * L, L)])
    pltpu.sync_copy(ne_v, ne_hbm.at[pl.ds(wid * L, L)])


_sc_all = pl.kernel(
    _sc_body,
    out_type=[
        jax.ShapeDtypeStruct((NW * L,), jnp.float32),
        jax.ShapeDtypeStruct((NW * L,), jnp.float32),
        jax.ShapeDtypeStruct((NW * L,), jnp.int32),
    ],
    mesh=plsc.VectorSubcoreMesh(core_axis_name="c", subcore_axis_name="s"),
    compiler_params=pltpu.CompilerParams(needs_layout_passes=False),
    scratch_types=[
        pltpu.VMEM((E,), jnp.float32),
        pltpu.VMEM((E,), jnp.float32),
        pltpu.VMEM((TABP,), jnp.float32),
        pltpu.VMEM((TABP,), jnp.float32),
        pltpu.VMEM((TABP,), jnp.int32),
        pltpu.VMEM((2 * DPW,), jnp.float32),
        pltpu.VMEM((D,), jnp.float32),
        pltpu.VMEM((D,), jnp.float32),
        pltpu.VMEM((DPW, D), jnp.float32),
        pltpu.VMEM((DPW, D), jnp.float32),
        pltpu.VMEM((L,), jnp.float32),
        pltpu.VMEM((L,), jnp.float32),
        pltpu.VMEM((L,), jnp.float32),
        pltpu.VMEM((L,), jnp.int32),
        pltpu.SemaphoreType.DMA,
        pltpu.SemaphoreType.DMA,
    ],
)


def kernel(events_x, events_y, calib_center, precompute_grid,
           pairwise_dists_mask, pairwise_dists, correction):
    fex = events_x.astype(jnp.float32)
    fey = events_y.astype(jnp.float32)
    t0 = precompute_grid[:, :, 0].reshape(-1)
    t1 = precompute_grid[:, :, 1].reshape(-1)
    nz = (t0 != 0).astype(jnp.int32) + (t1 != 0).astype(jnp.int32)
    ii = jnp.arange(G, dtype=jnp.int32)
    interior = jnp.logical_and(ii >= 1, ii <= G - 2)
    vic = jnp.logical_and(interior[:, None], interior[None, :])
    ct = nz + (vic.reshape(-1).astype(jnp.int32) << 16)
    t0p = jnp.pad(t0, (0, TABP - TAB))
    t1p = jnp.pad(t1, (0, TABP - TAB))
    ctp = jnp.pad(ct, (0, TABP - TAB))
    corrv = jnp.full((L,), correction, jnp.float32)
    ev = jnp.stack([fex, fey])

    new0, new1, ne = _sc_all(
        ev, t0p, t1p, ctp, calib_center.reshape(2 * D),
        calib_center[:, 0], calib_center[:, 1],
        pairwise_dists_mask, pairwise_dists, corrv)
    new0 = new0.reshape(NW, L)[:, :DPW].reshape(D)
    new1 = new1.reshape(NW, L)[:, :DPW].reshape(D)
    ne = ne.reshape(NW, L)[:, :DPW].reshape(D)
    calib_out = jnp.stack([new0, new1], axis=1)
    return (calib_out, ne)
